# Initial kernel scaffold; baseline (speedup 1.0000x reference)
import jax
import jax.numpy as jnp
from jax.experimental import pallas as pl


def kernel(p, x):
    raise NotImplementedError("write your pallas kernel here")


# speedup vs baseline: 4.6025x; 4.6025x over previous
import functools

import jax
import jax.numpy as jnp
from jax import lax
from jax.experimental import pallas as pl
from jax.experimental.pallas import tpu as pltpu
from jax.experimental.pallas import tpu_sc as plsc

B = 16
N = 16384
C = 64
S = 1024
L = 16
CHUNKS = N // L
UNROLL = 8

BIG_I32 = 2**30


def _xgather(xflat_hbm, xidx, xg, sem):
    cps = [pltpu.async_copy(xflat_hbm.at[xidx.at[q]],
                            xg.at[pl.ds(q * 128, 128)], sem)
           for q in range(8)]
    for cp in cps:
        cp.wait()


def _fps_body(p3_hbm, xflat_hbm, p3_out, xs_out,
              px, py, pz, dist, idxb, psx, psy, psz, xidx, xg, sem):
    c = lax.axis_index("c")
    s = lax.axis_index("s")
    b = c * 8 + s

    @pl.when(s < 8)
    def _():
        pltpu.sync_copy(p3_hbm.at[pl.ds((b * 3 + 0) * N, N)], px)
        pltpu.sync_copy(p3_hbm.at[pl.ds((b * 3 + 1) * N, N)], py)
        pltpu.sync_copy(p3_hbm.at[pl.ds((b * 3 + 2) * N, N)], pz)

        inf16 = jnp.full((L,), jnp.inf, dtype=jnp.float32)

        def init_body(j, _):
            base = j * (L * UNROLL)
            for u in range(UNROLL):
                dist[pl.ds(base + u * L, L)] = inf16
            return 0

        lax.fori_loop(0, CHUNKS // UNROLL, init_body, 0)

        lane = lax.iota(jnp.int32, L)
        lane0 = lane == 0

        def fps_iter(si, f):
            fv = jnp.full((L,), f, dtype=jnp.int32)
            plsc.store_scatter(idxb, [jnp.full((L,), si, dtype=jnp.int32)],
                               fv, mask=lane0)
            cx = plsc.load_gather(px, [fv])
            cy = plsc.load_gather(py, [fv])
            cz = plsc.load_gather(pz, [fv])

            def chunk_body(j, carry):
                vmax, vidx = carry
                base = j * (L * UNROLL)
                for u in range(UNROLL):
                    o = base + u * L
                    dx = px[pl.ds(o, L)] - cx
                    dy = py[pl.ds(o, L)] - cy
                    dz = pz[pl.ds(o, L)] - cz
                    d = dx * dx + dy * dy + dz * dz
                    nd = jnp.minimum(dist[pl.ds(o, L)], d)
                    dist[pl.ds(o, L)] = nd
                    gt = nd > vmax
                    vmax = jnp.where(gt, nd, vmax)
                    vidx = jnp.where(gt, o + lane, vidx)
                return vmax, vidx

            vmax0 = jnp.full((L,), -1.0, dtype=jnp.float32)
            vidx0 = jnp.zeros((L,), dtype=jnp.int32)
            vmax, vidx = lax.fori_loop(0, CHUNKS // UNROLL, chunk_body,
                                       (vmax0, vidx0))
            m = jnp.max(vmax)
            cand = jnp.where(vmax == m, vidx, BIG_I32)
            return jnp.min(cand)

        lax.fori_loop(0, S, fps_iter, jnp.int32(0))

        def ps_body(j, _):
            base = j * L
            iv = idxb[pl.ds(base, L)]
            psx[pl.ds(base, L)] = plsc.load_gather(px, [iv])
            psy[pl.ds(base, L)] = plsc.load_gather(py, [iv])
            psz[pl.ds(base, L)] = plsc.load_gather(pz, [iv])
            return 0

        lax.fori_loop(0, S // L, ps_body, 0)
        pltpu.sync_copy(psx, p3_out.at[pl.ds((b * 3 + 0) * S, S)])
        pltpu.sync_copy(psy, p3_out.at[pl.ds((b * 3 + 1) * S, S)])
        pltpu.sync_copy(psz, p3_out.at[pl.ds((b * 3 + 2) * S, S)])

        def chan_body(ci, _):
            off = (b * C + ci) * N

            def bld(k, _):
                q = k // 8
                r = k - q * 8
                xidx[q, pl.ds(r * L, L)] = idxb[pl.ds(k * L, L)] + off
                return 0

            lax.fori_loop(0, S // L, bld, 0)

            _xgather(xflat_hbm, xidx, xg, sem)
            pltpu.sync_copy(xg, xs_out.at[pl.ds((b * C + ci) * S, S)])
            return 0

        lax.fori_loop(0, C, chan_body, 0)


@jax.jit
def kernel(p, x):
    p3 = jnp.transpose(p, (0, 2, 1)).reshape(B * 3 * N)
    xflat = x.reshape(B * C * N)

    mesh = plsc.VectorSubcoreMesh(core_axis_name="c", subcore_axis_name="s",
                                  num_cores=2, num_subcores=16)
    fps = pl.kernel(
        _fps_body,
        out_type=(
            jax.ShapeDtypeStruct((B * 3 * S,), jnp.float32),
            jax.ShapeDtypeStruct((B * C * S,), jnp.float32),
        ),
        mesh=mesh,
        compiler_params=pltpu.CompilerParams(needs_layout_passes=False),
        scratch_types=[
            pltpu.VMEM((N,), jnp.float32),
            pltpu.VMEM((N,), jnp.float32),
            pltpu.VMEM((N,), jnp.float32),
            pltpu.VMEM((N,), jnp.float32),
            pltpu.VMEM((S,), jnp.int32),
            pltpu.VMEM((S,), jnp.float32),
            pltpu.VMEM((S,), jnp.float32),
            pltpu.VMEM((S,), jnp.float32),
            pltpu.VMEM((8, 128), jnp.int32),
            pltpu.VMEM((S,), jnp.float32),
            pltpu.SemaphoreType.DMA,
        ],
    )
    p3_s, x_s = fps(p3, xflat)
    p_s = jnp.transpose(p3_s.reshape(B, 3, S), (0, 2, 1))
    return (p_s, x_s.reshape(B, C, S))
